# baseline (speedup 1.0000x reference)
import functools

import jax
import jax.numpy as jnp
from jax import lax
from jax.experimental import pallas as pl
from jax.experimental.pallas import tpu as pltpu
from jax.experimental.pallas import tpu_sc as plsc

D = 32
NC, NS = 2, 16
NW = NC * NS
RG = 8
NBUF = 2


@jax.jit
def _embed(idx, table):
    n, s = idx.shape
    rows_w = n // NW
    steps = rows_w // RG
    segs = [(o, min(128, s - o)) for o in range(0, s, 128)]
    mesh = plsc.VectorSubcoreMesh(
        core_axis_name="c", subcore_axis_name="s", num_cores=NC, num_subcores=NS
    )

    @functools.partial(
        pl.kernel,
        out_type=jax.ShapeDtypeStruct((n, s, D), jnp.float32),
        mesh=mesh,
        scratch_types=[
            pltpu.VMEM((rows_w, s), jnp.int32),
            pltpu.VMEM((NBUF, RG, s, D), jnp.float32),
            [pltpu.SemaphoreType.DMA] * NBUF,
            [pltpu.SemaphoreType.DMA] * NBUF,
        ],
        compiler_params=pltpu.CompilerParams(use_tc_tiling_on_sc=False),
    )
    def emb(idx_hbm, table_hbm, out_hbm, idx_v, rows_v, gsems, ssems):
        wid = lax.axis_index("s") * NC + lax.axis_index("c")
        base = pl.multiple_of(wid * rows_w, RG)

        pltpu.sync_copy(idx_hbm.at[pl.ds(base, rows_w), :], idx_v)

        def fire(step, b):
            r0 = pl.multiple_of(step * RG, RG)
            for r in range(RG):
                for (o, ln) in segs:
                    pltpu.async_copy(
                        table_hbm.at[idx_v.at[r0 + r, pl.ds(o, ln)]],
                        rows_v.at[b, r, pl.ds(o, ln), :],
                        gsems[b],
                    )

        def drain(sem, b):
            pltpu.make_async_copy(
                out_hbm.at[pl.ds(0, RG), :, :], rows_v.at[b], sem
            ).wait()

        for b in range(NBUF):
            fire(b, b)

        def body(g, carry):
            for b in range(NBUF):
                step = g * NBUF + b

                @pl.when(step < steps)
                def _():
                    drain(gsems[b], b)
                    off = pl.multiple_of(base + step * RG, RG)
                    pltpu.async_copy(
                        rows_v.at[b], out_hbm.at[pl.ds(off, RG), :, :], ssems[b]
                    )

                    @pl.when(step + NBUF < steps)
                    def _():
                        drain(ssems[b], b)
                        fire(step + NBUF, b)

            return carry

        lax.fori_loop(0, (steps + NBUF - 1) // NBUF, body, 0)
        for b in range(NBUF):
            drain(ssems[b], b)

    return emb(idx, table)


def kernel(tensor, weight):
    return _embed(tensor.astype(jnp.int32), weight)
